# baseline (device time: 17872 ns/iter reference)
import jax
import jax.numpy as jnp
from jax import lax
from jax.experimental import pallas as pl
from jax.experimental.pallas import tpu as pltpu

N_DEV = 4
B, SQ, SKV, D_MODEL = 2, 256, 256, 512
H_LOC, DH = 4, 64
BLK = 64
QROWS = (B * SQ) // N_DEV


def kernel(x, Wq, K_ext, V_ext, Wo):
    i = lax.axis_index("i")
    K = lax.dynamic_slice_in_dim(K_ext, i * H_LOC, H_LOC, axis=2).reshape(
        B, SKV, H_LOC * DH)
    V = lax.dynamic_slice_in_dim(V_ext, i * H_LOC, H_LOC, axis=2).reshape(
        B, SKV, H_LOC * DH)

    def body(x_ref, wq_ref, k_scr, v_scr, wo_ref, out_ref,
             partial_ref, rs_recv, red_ref,
             rs_send_sems, rs_recv_sems, ag_send_sems, ag_recv_sems):
        my = lax.axis_index("i")

        barrier = pltpu.get_barrier_semaphore()
        for j in range(1, N_DEV):
            pl.semaphore_signal(barrier, inc=1,
                                device_id=(lax.rem(my + j, N_DEV),),
                                device_id_type=pl.DeviceIdType.MESH)
        pl.semaphore_wait(barrier, N_DEV - 1)

        wq = wq_ref[...].astype(jnp.bfloat16)
        wo = wo_ref[...].astype(jnp.bfloat16)

        def make_mask(skv, r0):
            rblk = (r0 + lax.broadcasted_iota(jnp.int32, (QROWS, skv), 0)) // BLK
            cblk = lax.broadcasted_iota(jnp.int32, (QROWS, skv), 1) // BLK
            return cblk <= rblk

        mask_even = make_mask(QROWS, 0)
        mask_odd = make_mask(SKV, QROWS)

        rs_rdmas = []
        for q in range(N_DEV):
            b, r0 = q // 2, (q % 2) * QROWS
            skv = QROWS if q % 2 == 0 else SKV
            mask = mask_even if q % 2 == 0 else mask_odd
            xb = x_ref[b, r0:r0 + QROWS, :].astype(jnp.bfloat16)
            qp = jnp.dot(xb, wq, preferred_element_type=jnp.float32)
            ctx = []
            for h in range(H_LOC):
                qh = qp[:, h * DH:(h + 1) * DH].astype(jnp.bfloat16)
                kh = k_scr[b, :skv, h * DH:(h + 1) * DH].astype(jnp.bfloat16)
                vh = v_scr[b, :skv, h * DH:(h + 1) * DH].astype(jnp.bfloat16)
                s = lax.dot_general(qh, kh, (((1,), (1,)), ((), ())),
                                    preferred_element_type=jnp.float32) * 0.125
                w = jnp.exp(jnp.where(mask, s, -1e9))
                denom = jnp.sum(w, axis=-1, keepdims=True)
                pv = jnp.dot(w.astype(jnp.bfloat16), vh,
                             preferred_element_type=jnp.float32)
                ctx.append(pv / denom)
            ctx_all = jnp.concatenate(ctx, axis=1).astype(jnp.bfloat16)
            pr = jnp.dot(ctx_all, wo, preferred_element_type=jnp.float32)
            partial_ref[q] = pr.astype(jnp.bfloat16)
            rdma = pltpu.make_async_remote_copy(
                src_ref=partial_ref.at[q],
                dst_ref=rs_recv.at[my],
                send_sem=rs_send_sems.at[q],
                recv_sem=rs_recv_sems.at[my],
                device_id=(q,),
                device_id_type=pl.DeviceIdType.MESH,
            )
            rdma.start()
            rs_rdmas.append(rdma)

        for s in range(N_DEV):
            pltpu.make_async_remote_copy(
                src_ref=rs_recv.at[s], dst_ref=rs_recv.at[s],
                send_sem=rs_send_sems.at[s], recv_sem=rs_recv_sems.at[s],
                device_id=(s,), device_id_type=pl.DeviceIdType.MESH,
            ).wait_recv()

        acc = (rs_recv[0].astype(jnp.float32) + rs_recv[1].astype(jnp.float32)
               + rs_recv[2].astype(jnp.float32) + rs_recv[3].astype(jnp.float32))
        red = acc.astype(jnp.bfloat16)
        red_ref[...] = red
        my_b = my // 2
        my_row = (my % 2) * QROWS
        out_ref[my_b, pl.ds(my_row, QROWS), :] = red

        ag_rdmas = []
        for j in range(1, N_DEV):
            rdma = pltpu.make_async_remote_copy(
                src_ref=red_ref,
                dst_ref=out_ref.at[my_b, pl.ds(my_row, QROWS), :],
                send_sem=ag_send_sems.at[j - 1],
                recv_sem=ag_recv_sems.at[N_DEV - j],
                device_id=(lax.rem(my + j, N_DEV),),
                device_id_type=pl.DeviceIdType.MESH,
            )
            rdma.start()
            ag_rdmas.append(rdma)

        for k in range(1, N_DEV):
            pltpu.make_async_remote_copy(
                src_ref=red_ref,
                dst_ref=out_ref.at[0, pl.ds(0, QROWS), :],
                send_sem=ag_send_sems.at[0], recv_sem=ag_recv_sems.at[k],
                device_id=(my,), device_id_type=pl.DeviceIdType.MESH,
            ).wait_recv()

        for rdma in rs_rdmas + ag_rdmas:
            rdma.wait_send()

    return pl.pallas_call(
        body,
        out_shape=jax.ShapeDtypeStruct((B, SQ, D_MODEL), jnp.bfloat16),
        in_specs=[
            pl.BlockSpec(memory_space=pltpu.VMEM),
            pl.BlockSpec(memory_space=pltpu.VMEM),
            pl.BlockSpec(memory_space=pltpu.VMEM),
            pl.BlockSpec(memory_space=pltpu.VMEM),
            pl.BlockSpec(memory_space=pltpu.VMEM),
        ],
        out_specs=pl.BlockSpec(memory_space=pltpu.VMEM),
        scratch_shapes=[
            pltpu.VMEM((N_DEV, QROWS, D_MODEL), jnp.bfloat16),
            pltpu.VMEM((N_DEV, QROWS, D_MODEL), jnp.bfloat16),
            pltpu.VMEM((QROWS, D_MODEL), jnp.bfloat16),
            pltpu.SemaphoreType.DMA((N_DEV,)),
            pltpu.SemaphoreType.DMA((N_DEV,)),
            pltpu.SemaphoreType.DMA((N_DEV - 1,)),
            pltpu.SemaphoreType.DMA((N_DEV,)),
        ],
        compiler_params=pltpu.CompilerParams(collective_id=0),
    )(x, Wq, K, V, Wo)


# device time: 16801 ns/iter; 1.0637x vs baseline; 1.0637x over previous
import jax
import jax.numpy as jnp
from jax import lax
from jax.experimental import pallas as pl
from jax.experimental.pallas import tpu as pltpu

N_DEV = 4
B, SQ, SKV, D_MODEL = 2, 256, 256, 512
H_LOC, DH = 4, 64
BLK = 64
QROWS = (B * SQ) // N_DEV
HROWS = QROWS // 2


def kernel(x, Wq, K_ext, V_ext, Wo):
    i = lax.axis_index("i")
    K = lax.dynamic_slice_in_dim(K_ext, i * H_LOC, H_LOC, axis=2).reshape(
        B, SKV, H_LOC * DH)
    V = lax.dynamic_slice_in_dim(V_ext, i * H_LOC, H_LOC, axis=2).reshape(
        B, SKV, H_LOC * DH)

    def body(x_ref, wq_ref, k_scr, v_scr, wo_ref, out_ref,
             partial_ref, rs_recv, red_ref,
             rs_send_sems, rs_recv_sems, ag_send_sems, ag_recv_sems):
        my = lax.axis_index("i")

        wq = wq_ref[...].astype(jnp.bfloat16)
        wo = wo_ref[...].astype(jnp.bfloat16)
        cols_blk = lax.broadcasted_iota(jnp.int32, (QROWS, SKV), 1) // BLK
        rows_iota = lax.broadcasted_iota(jnp.int32, (QROWS, SKV), 0)

        def compute_quarter(qtr):
            b = qtr // 2
            r0 = (qtr % 2) * QROWS
            xb = x_ref[b, pl.ds(r0, QROWS), :].astype(jnp.bfloat16)
            qp = jnp.dot(xb, wq, preferred_element_type=jnp.float32)
            mask = cols_blk <= (r0 + rows_iota) // BLK
            ctx = []
            for h in range(H_LOC):
                qh = qp[:, h * DH:(h + 1) * DH].astype(jnp.bfloat16)
                kh = k_scr[b, :, h * DH:(h + 1) * DH].astype(jnp.bfloat16)
                vh = v_scr[b, :, h * DH:(h + 1) * DH].astype(jnp.bfloat16)
                s = lax.dot_general(qh, kh, (((1,), (1,)), ((), ())),
                                    preferred_element_type=jnp.float32) * 0.125
                w = jnp.exp(jnp.where(mask, s, -1e9))
                denom = jnp.sum(w, axis=-1, keepdims=True)
                pv = jnp.dot(w.astype(jnp.bfloat16), vh,
                             preferred_element_type=jnp.float32)
                ctx.append(pv / denom)
            ctx_all = jnp.concatenate(ctx, axis=1).astype(jnp.bfloat16)
            return jnp.dot(ctx_all, wo, preferred_element_type=jnp.float32)

        rs_rdmas = []
        for j in range(1, N_DEV):
            qtr = lax.rem(my + j, N_DEV)
            partial_ref[j - 1] = compute_quarter(qtr).astype(jnp.bfloat16)
            if j == 1:
                barrier = pltpu.get_barrier_semaphore()
                for n in range(1, N_DEV):
                    pl.semaphore_signal(barrier, inc=1,
                                        device_id=(lax.rem(my + n, N_DEV),),
                                        device_id_type=pl.DeviceIdType.MESH)
                pl.semaphore_wait(barrier, N_DEV - 1)
            for c in range(2):
                lo = c * HROWS
                rdma = pltpu.make_async_remote_copy(
                    src_ref=partial_ref.at[j - 1, pl.ds(lo, HROWS), :],
                    dst_ref=rs_recv.at[N_DEV - j, pl.ds(lo, HROWS), :],
                    send_sem=rs_send_sems.at[c, j - 1],
                    recv_sem=rs_recv_sems.at[c, N_DEV - j],
                    device_id=(qtr,),
                    device_id_type=pl.DeviceIdType.MESH,
                )
                rdma.start()
                rs_rdmas.append(rdma)

        own_bf = compute_quarter(my).astype(jnp.bfloat16)

        my_b = my // 2
        my_row = (my % 2) * QROWS
        ag_rdmas = []
        for c in range(2):
            lo = c * HROWS
            for k in range(1, N_DEV):
                pltpu.make_async_remote_copy(
                    src_ref=rs_recv.at[k, pl.ds(lo, HROWS), :],
                    dst_ref=rs_recv.at[k, pl.ds(lo, HROWS), :],
                    send_sem=rs_send_sems.at[0, 0],
                    recv_sem=rs_recv_sems.at[c, k],
                    device_id=(my,), device_id_type=pl.DeviceIdType.MESH,
                ).wait_recv()
            red = ((own_bf[lo:lo + HROWS] + rs_recv[1, lo:lo + HROWS])
                   + (rs_recv[2, lo:lo + HROWS] + rs_recv[3, lo:lo + HROWS]))
            out_ref[my_b, pl.ds(my_row + lo, HROWS), :] = red
            for j in range(1, N_DEV):
                rdma = pltpu.make_async_remote_copy(
                    src_ref=out_ref.at[my_b, pl.ds(my_row + lo, HROWS), :],
                    dst_ref=out_ref.at[my_b, pl.ds(my_row + lo, HROWS), :],
                    send_sem=ag_send_sems.at[c, j - 1],
                    recv_sem=ag_recv_sems.at[c, N_DEV - j],
                    device_id=(lax.rem(my + j, N_DEV),),
                    device_id_type=pl.DeviceIdType.MESH,
                )
                rdma.start()
                ag_rdmas.append(rdma)

        for c in range(2):
            for k in range(1, N_DEV):
                pltpu.make_async_remote_copy(
                    src_ref=red_ref.at[pl.ds(0, HROWS), :],
                    dst_ref=out_ref.at[0, pl.ds(0, HROWS), :],
                    send_sem=ag_send_sems.at[0, 0],
                    recv_sem=ag_recv_sems.at[c, k],
                    device_id=(my,), device_id_type=pl.DeviceIdType.MESH,
                ).wait_recv()

        for rdma in rs_rdmas + ag_rdmas:
            rdma.wait_send()

    return pl.pallas_call(
        body,
        out_shape=jax.ShapeDtypeStruct((B, SQ, D_MODEL), jnp.bfloat16),
        in_specs=[
            pl.BlockSpec(memory_space=pltpu.VMEM),
            pl.BlockSpec(memory_space=pltpu.VMEM),
            pl.BlockSpec(memory_space=pltpu.VMEM),
            pl.BlockSpec(memory_space=pltpu.VMEM),
            pl.BlockSpec(memory_space=pltpu.VMEM),
        ],
        out_specs=pl.BlockSpec(memory_space=pltpu.VMEM),
        scratch_shapes=[
            pltpu.VMEM((N_DEV - 1, QROWS, D_MODEL), jnp.bfloat16),
            pltpu.VMEM((N_DEV, QROWS, D_MODEL), jnp.bfloat16),
            pltpu.VMEM((QROWS, D_MODEL), jnp.bfloat16),
            pltpu.SemaphoreType.DMA((2, N_DEV - 1)),
            pltpu.SemaphoreType.DMA((2, N_DEV)),
            pltpu.SemaphoreType.DMA((2, N_DEV - 1)),
            pltpu.SemaphoreType.DMA((2, N_DEV)),
        ],
        compiler_params=pltpu.CompilerParams(collective_id=0),
    )(x, Wq, K, V, Wo)
